# Initial kernel scaffold; baseline (speedup 1.0000x reference)
import jax
import jax.numpy as jnp
from jax.experimental import pallas as pl


def kernel(inputs, table):
    raise NotImplementedError("write your pallas kernel here")


# speedup vs baseline: 10.8167x; 10.8167x over previous
import functools

import jax
import jax.numpy as jnp
from jax import lax
from jax.experimental import pallas as pl
from jax.experimental.pallas import tpu as pltpu
from jax.experimental.pallas import tpu_sc as plsc

V_DIM = 100000
EMB = 128
BATCH = 16384
HIST = 50

NC, NS = 2, 16
NW = NC * NS
ROWS_PER_W = BATCH // NW
CHUNK = 8
NCHUNK = ROWS_PER_W // CHUNK
IDX_ROW = 100
GPC = CHUNK * HIST // IDX_ROW
LANES = 16
COLS = EMB // LANES
SCALE = 1.0 / HIST

_mesh = plsc.VectorSubcoreMesh(core_axis_name="c", subcore_axis_name="s")


@functools.partial(
    pl.kernel,
    out_type=jax.ShapeDtypeStruct((BATCH, EMB), jnp.float32),
    mesh=_mesh,
    scratch_types=[
        pltpu.VMEM((ROWS_PER_W * HIST // IDX_ROW, IDX_ROW), jnp.int32),
        pltpu.VMEM((CHUNK * HIST, EMB), jnp.float32),
        pltpu.VMEM((CHUNK, EMB), jnp.float32),
        pltpu.SemaphoreType.DMA,
    ],
)
def _cbow_sc(table_hbm, idx_hbm, out_hbm, idx_v, rows_v, outb, sem):
    wid = lax.axis_index("s") * NC + lax.axis_index("c")
    idx_rows_per_w = ROWS_PER_W * HIST // IDX_ROW
    pltpu.sync_copy(idx_hbm.at[pl.ds(wid * idx_rows_per_w, idx_rows_per_w), :],
                    idx_v)

    @pl.loop(0, NCHUNK)
    def chunk(i):
        cps = [
            pltpu.async_copy(
                table_hbm.at[idx_v.at[i * GPC + g]],
                rows_v.at[pl.ds(g * IDX_ROW, IDX_ROW), :],
                sem,
            )
            for g in range(GPC)
        ]
        for cp in cps:
            cp.wait()
        for b in range(CHUNK):
            def hbody(h, accs):
                r = b * HIST + h
                return tuple(accs[c] + rows_v[r, pl.ds(c * LANES, LANES)]
                             for c in range(COLS))
            accs = lax.fori_loop(
                0, HIST, hbody,
                tuple(jnp.zeros((LANES,), jnp.float32) for _ in range(COLS)))
            for c in range(COLS):
                outb[b, pl.ds(c * LANES, LANES)] = accs[c] * SCALE
        pltpu.sync_copy(outb,
                        out_hbm.at[pl.ds(wid * ROWS_PER_W + i * CHUNK, CHUNK), :])


def kernel(inputs, table):
    idx = inputs.astype(jnp.int32).reshape(BATCH * HIST // IDX_ROW, IDX_ROW)
    return _cbow_sc(table, idx)


# speedup vs baseline: 20.4282x; 1.8886x over previous
import functools

import jax
import jax.numpy as jnp
from jax import lax
from jax.experimental import pallas as pl
from jax.experimental.pallas import tpu as pltpu
from jax.experimental.pallas import tpu_sc as plsc

V_DIM = 100000
EMB = 128
BATCH = 16384
HIST = 50

NC, NS = 2, 16
NW = NC * NS
ROWS_PER_W = BATCH // NW
CHUNK = 4
NCHUNK = ROWS_PER_W // CHUNK
IDX_ROW = 100
GPC = CHUNK * HIST // IDX_ROW
LANES = 16
COLS = EMB // LANES
SCALE = 1.0 / HIST

_mesh = plsc.VectorSubcoreMesh(core_axis_name="c", subcore_axis_name="s")


@functools.partial(
    pl.kernel,
    out_type=jax.ShapeDtypeStruct((BATCH, EMB), jnp.float32),
    mesh=_mesh,
    scratch_types=[
        pltpu.VMEM((ROWS_PER_W * HIST // IDX_ROW, IDX_ROW), jnp.int32),
        pltpu.VMEM((2, CHUNK * HIST, EMB), jnp.float32),
        pltpu.VMEM((CHUNK, EMB), jnp.float32),
        pltpu.SemaphoreType.DMA,
        pltpu.SemaphoreType.DMA,
    ],
)
def _cbow_sc(table_hbm, idx_hbm, out_hbm, idx_v, rows_v, outb, sem0, sem1):
    wid = lax.axis_index("s") * NC + lax.axis_index("c")
    sems = (sem0, sem1)
    idx_rows_per_w = ROWS_PER_W * HIST // IDX_ROW
    pltpu.sync_copy(idx_hbm.at[pl.ds(wid * idx_rows_per_w, idx_rows_per_w), :],
                    idx_v)

    def gathers(i, b):
        return [
            pltpu.make_async_copy(
                table_hbm.at[idx_v.at[i * GPC + g]],
                rows_v.at[b, pl.ds(g * IDX_ROW, IDX_ROW), :],
                sems[b],
            )
            for g in range(GPC)
        ]

    def fire(i, b):
        for cp in gathers(i, b):
            cp.start()

    def drain(i, b):
        for cp in gathers(i, b):
            cp.wait()

    def compute(i, b):
        for r0 in range(CHUNK):
            def hbody(h, accs):
                r = r0 * HIST + h
                return tuple(accs[c] + rows_v[b, r, pl.ds(c * LANES, LANES)]
                             for c in range(COLS))
            accs = lax.fori_loop(
                0, HIST, hbody,
                tuple(jnp.zeros((LANES,), jnp.float32) for _ in range(COLS)))
            for c in range(COLS):
                outb[r0, pl.ds(c * LANES, LANES)] = accs[c] * SCALE
        pltpu.sync_copy(outb,
                        out_hbm.at[pl.ds(wid * ROWS_PER_W + i * CHUNK, CHUNK), :])

    fire(0, 0)
    fire(1, 1)

    @pl.loop(0, NCHUNK, step=2)
    def chunk(j):
        for b in range(2):
            i = j + b
            drain(i, b)
            nxt = i + 2
            @pl.when(nxt < NCHUNK)
            def _():
                fire(nxt, b)
            compute(i, b)


def kernel(inputs, table):
    idx = inputs.astype(jnp.int32).reshape(BATCH * HIST // IDX_ROW, IDX_ROW)
    return _cbow_sc(table, idx)
